# baseline (speedup 1.0000x reference)
import functools

import jax
import jax.numpy as jnp
from jax import lax
from jax.experimental import pallas as pl
from jax.experimental.pallas import tpu as pltpu
from jax.experimental.pallas import tpu_sc as plsc


def kernel(x, pos_table):
    B, S, D = x.shape
    dtype = pos_table.dtype

    info = plsc.get_sparse_core_info()
    NC, NS = info.num_cores, info.num_subcores
    NW = NC * NS
    rows_per_w = S // NW
    CHUNK = 32
    nchunks = rows_per_w // CHUNK

    mesh = plsc.VectorSubcoreMesh(core_axis_name="c", subcore_axis_name="s")

    NBUF = 3

    @functools.partial(
        pl.kernel,
        mesh=mesh,
        out_type=jax.ShapeDtypeStruct((B, S, D), dtype),
        scratch_types=[
            pltpu.VMEM_SHARED((NS, NBUF, CHUNK, D), dtype),
            pltpu.SemaphoreType.DMA,
            pltpu.SemaphoreType.DMA,
        ],
    )
    def broadcast_rows(table_hbm, out_hbm, shared, lsem, ssem):
        sid = lax.axis_index("s")
        wid = sid * NC + lax.axis_index("c")
        row0 = wid * rows_per_w
        buf = shared.at[sid]

        loads = [None] * nchunks
        stores = [None] * nchunks

        def start_load(c):
            loads[c] = pltpu.async_copy(
                table_hbm.at[pl.ds(row0 + c * CHUNK, CHUNK)], buf.at[c % NBUF], lsem
            )

        for n in range(min(NBUF, nchunks)):
            start_load(n)
        for c in range(nchunks):
            if c >= 1:
                for h in stores[c - 1]:
                    h.wait()
                n = (c - 1) + NBUF
                if n < nchunks:
                    start_load(n)
            loads[c].wait()
            stores[c] = [
                pltpu.async_copy(
                    buf.at[c % NBUF], out_hbm.at[b, pl.ds(row0 + c * CHUNK, CHUNK)], ssem
                )
                for b in range(B)
            ]
        for h in stores[nchunks - 1]:
            h.wait()

    return broadcast_rows(pos_table)


# speedup vs baseline: 1.2107x; 1.2107x over previous
import functools

import jax
import jax.numpy as jnp
from jax import lax
from jax.experimental import pallas as pl
from jax.experimental.pallas import tpu as pltpu
from jax.experimental.pallas import tpu_sc as plsc


def kernel(x, pos_table):
    B, S, D = x.shape
    dtype = pos_table.dtype

    info = plsc.get_sparse_core_info()
    NC, NS = info.num_cores, info.num_subcores
    NW = NC * NS
    rows_per_w = S // NW
    CHUNK = 16
    NT = 5
    NSP = 3
    assert (NT + NSP) * CHUNK == rows_per_w
    TBUF = 3

    mesh = plsc.VectorSubcoreMesh(core_axis_name="c", subcore_axis_name="s")

    @functools.partial(
        pl.kernel,
        mesh=mesh,
        out_type=jax.ShapeDtypeStruct((B, S, D), dtype),
        scratch_types=[
            pltpu.VMEM((TBUF, CHUNK, D), dtype),
            pltpu.VMEM_SHARED((NS, NSP, CHUNK, D), dtype),
            pltpu.SemaphoreType.DMA,
            pltpu.SemaphoreType.DMA,
            pltpu.SemaphoreType.DMA,
            pltpu.SemaphoreType.DMA,
        ],
    )
    def broadcast_rows(table_hbm, out_hbm, tbuf, shared, tl, ts, sl, ss):
        sid = lax.axis_index("s")
        wid = sid * NC + lax.axis_index("c")
        row0 = wid * rows_per_w
        sbuf = shared.at[sid]

        t_off = [row0 + c * CHUNK for c in range(NT)]
        s_off = [row0 + (NT + c) * CHUNK for c in range(NSP)]

        t_loads, t_stores = [None] * NT, [None] * NT
        s_loads, s_stores = [None] * NSP, [None] * NSP

        def t_load(c):
            t_loads[c] = pltpu.async_copy(
                table_hbm.at[pl.ds(t_off[c], CHUNK)], tbuf.at[c % TBUF], tl
            )

        def t_store(c):
            t_stores[c] = [
                pltpu.async_copy(
                    tbuf.at[c % TBUF], out_hbm.at[b, pl.ds(t_off[c], CHUNK)], ts
                )
                for b in range(B)
            ]

        def s_store(c):
            s_stores[c] = [
                pltpu.async_copy(
                    sbuf.at[c], out_hbm.at[b, pl.ds(s_off[c], CHUNK)], ss
                )
                for b in range(B)
            ]

        for c in range(NSP):
            s_loads[c] = pltpu.async_copy(
                table_hbm.at[pl.ds(s_off[c], CHUNK)], sbuf.at[c], sl
            )
        for c in range(min(TBUF, NT)):
            t_load(c)

        for c in range(NT):
            if c >= 1:
                for h in t_stores[c - 1]:
                    h.wait()
                n = (c - 1) + TBUF
                if n < NT:
                    t_load(n)
            t_loads[c].wait()
            t_store(c)
            if c < NSP:
                s_loads[c].wait()
                s_store(c)
        for h in t_stores[NT - 1]:
            h.wait()
        for c in range(NSP):
            for h in s_stores[c]:
                h.wait()

    return broadcast_rows(pos_table)
